# baseline (speedup 1.0000x reference)
import functools

import jax
import jax.numpy as jnp
from jax import lax
from jax.experimental import pallas as pl
from jax.experimental.pallas import tpu as pltpu
from jax.experimental.pallas import tpu_sc as plsc

BATCH = 16384
EMBED_DIM = 64
NUM_WORKERS = 32
B_PER_W = BATCH // NUM_WORKERS


def _dot_kernel(uid_hbm, iid_hbm, uemb_hbm, iemb_hbm, ibias_hbm, out_hbm,
                uidx_v, iidx_v, urows_v, irows_v, bias_v, out_v, sem):
    wid = lax.axis_index("s") * 2 + lax.axis_index("c")
    base = wid * B_PER_W

    pltpu.sync_copy(uid_hbm.at[pl.ds(base, B_PER_W)], uidx_v)
    pltpu.sync_copy(iid_hbm.at[pl.ds(base, B_PER_W)], iidx_v)

    cu = pltpu.async_copy(uemb_hbm.at[uidx_v], urows_v, sem)
    ci = pltpu.async_copy(iemb_hbm.at[iidx_v], irows_v, sem)
    cb = pltpu.async_copy(ibias_hbm.at[iidx_v], bias_v, sem)
    cu.wait()
    ci.wait()
    cb.wait()
    zeros16 = jnp.zeros((16,), jnp.int32)

    iota16 = lax.iota(jnp.int32, 16)

    def group(g, carry):
        rows = g * 16 + iota16
        accs = [jnp.zeros((16,), jnp.float32) for _ in range(4)]
        for d in range(EMBED_DIM):
            col = jnp.full((16,), d, jnp.int32)
            u = plsc.load_gather(urows_v, [rows, col])
            w = plsc.load_gather(irows_v, [rows, col])
            accs[d % 4] = accs[d % 4] + u * w
        total = (accs[0] + accs[1]) + (accs[2] + accs[3])
        b = plsc.load_gather(bias_v, [rows, zeros16])
        out_v[pl.ds(g * 16, 16)] = total + b
        return carry

    lax.fori_loop(0, B_PER_W // 16, group, 0)

    pltpu.sync_copy(out_v, out_hbm.at[pl.ds(base, B_PER_W)])


@jax.jit
def kernel(user_ids, item_ids, user_embed, item_embed, item_bias):
    mesh = plsc.VectorSubcoreMesh(core_axis_name="c", subcore_axis_name="s")
    run = functools.partial(
        pl.kernel,
        mesh=mesh,
        compiler_params=pltpu.CompilerParams(
            needs_layout_passes=False, use_tc_tiling_on_sc=False),
        out_type=jax.ShapeDtypeStruct((BATCH,), jnp.float32),
        scratch_types=[
            pltpu.VMEM((B_PER_W,), jnp.int32),
            pltpu.VMEM((B_PER_W,), jnp.int32),
            pltpu.VMEM((B_PER_W, EMBED_DIM), jnp.float32),
            pltpu.VMEM((B_PER_W, EMBED_DIM), jnp.float32),
            pltpu.VMEM((B_PER_W, 1), jnp.float32),
            pltpu.VMEM((B_PER_W,), jnp.float32),
            pltpu.SemaphoreType.DMA,
        ],
    )(_dot_kernel)
    return run(user_ids, item_ids, user_embed, item_embed, item_bias)


# speedup vs baseline: 1.1235x; 1.1235x over previous
import functools

import jax
import jax.numpy as jnp
from jax import lax
from jax.experimental import pallas as pl
from jax.experimental.pallas import tpu as pltpu
from jax.experimental.pallas import tpu_sc as plsc

BATCH = 16384
EMBED_DIM = 64
NUM_WORKERS = 32
B_PER_W = BATCH // NUM_WORKERS


def _dot_kernel(uid_hbm, iid_hbm, uemb_hbm, iemb_hbm, ibias_hbm, out_hbm,
                uidx_v, iidx_v, urows_v, irows_v, bias_v, out_v, sem):
    wid = lax.axis_index("s") * 2 + lax.axis_index("c")
    base = wid * B_PER_W

    pltpu.sync_copy(uid_hbm.at[pl.ds(base, B_PER_W)], uidx_v)
    pltpu.sync_copy(iid_hbm.at[pl.ds(base, B_PER_W)], iidx_v)

    cu = pltpu.async_copy(uemb_hbm.at[uidx_v], urows_v, sem)
    ci = pltpu.async_copy(iemb_hbm.at[iidx_v], irows_v, sem)
    cb = pltpu.async_copy(ibias_hbm.at[iidx_v], bias_v, sem)
    cu.wait()
    ci.wait()
    cb.wait()

    iota16 = lax.iota(jnp.int32, 16)
    zeros16 = jnp.zeros((16,), jnp.int32)

    def rows16(g, carry):
        out16 = jnp.zeros((16,), jnp.float32)
        for k in range(16):
            r = g * 16 + k
            u0 = urows_v[r, pl.ds(0, 16)]
            u1 = urows_v[r, pl.ds(16, 16)]
            u2 = urows_v[r, pl.ds(32, 16)]
            u3 = urows_v[r, pl.ds(48, 16)]
            w0 = irows_v[r, pl.ds(0, 16)]
            w1 = irows_v[r, pl.ds(16, 16)]
            w2 = irows_v[r, pl.ds(32, 16)]
            w3 = irows_v[r, pl.ds(48, 16)]
            acc = (u0 * w0 + u1 * w1) + (u2 * w2 + u3 * w3)
            s = jnp.sum(acc)
            out16 = jnp.where(iota16 == k, s, out16)
        b16 = plsc.load_gather(bias_v, [g * 16 + iota16, zeros16])
        out_v[pl.ds(g * 16, 16)] = out16 + b16
        return carry

    lax.fori_loop(0, B_PER_W // 16, rows16, 0)

    pltpu.sync_copy(out_v, out_hbm.at[pl.ds(base, B_PER_W)])


@jax.jit
def kernel(user_ids, item_ids, user_embed, item_embed, item_bias):
    mesh = plsc.VectorSubcoreMesh(core_axis_name="c", subcore_axis_name="s")
    run = functools.partial(
        pl.kernel,
        mesh=mesh,
        compiler_params=pltpu.CompilerParams(
            needs_layout_passes=False, use_tc_tiling_on_sc=False),
        out_type=jax.ShapeDtypeStruct((BATCH,), jnp.float32),
        scratch_types=[
            pltpu.VMEM((B_PER_W,), jnp.int32),
            pltpu.VMEM((B_PER_W,), jnp.int32),
            pltpu.VMEM((B_PER_W, EMBED_DIM), jnp.float32),
            pltpu.VMEM((B_PER_W, EMBED_DIM), jnp.float32),
            pltpu.VMEM((B_PER_W, 1), jnp.float32),
            pltpu.VMEM((B_PER_W,), jnp.float32),
            pltpu.SemaphoreType.DMA,
        ],
    )(_dot_kernel)
    return run(user_ids, item_ids, user_embed, item_embed, item_bias)


# speedup vs baseline: 1.1267x; 1.0028x over previous
import functools

import jax
import jax.numpy as jnp
from jax import lax
from jax.experimental import pallas as pl
from jax.experimental.pallas import tpu as pltpu
from jax.experimental.pallas import tpu_sc as plsc

BATCH = 16384
EMBED_DIM = 64
NUM_WORKERS = 32
B_PER_W = BATCH // NUM_WORKERS


def _dot_kernel(uid_hbm, iid_hbm, uemb_hbm, iemb_hbm, ibias_hbm, out_hbm,
                uidx_v, iidx_v, urows_v, irows_v, bias_v, out_v, sem):
    wid = lax.axis_index("s") * 2 + lax.axis_index("c")
    base = wid * B_PER_W

    pltpu.sync_copy(uid_hbm.at[pl.ds(base, B_PER_W)], uidx_v)
    pltpu.sync_copy(iid_hbm.at[pl.ds(base, B_PER_W)], iidx_v)

    n_ch = 4
    ch = B_PER_W // n_ch
    copies = []
    for c in range(n_ch):
        sl = pl.ds(c * ch, ch)
        copies.append(pltpu.async_copy(
            uemb_hbm.at[uidx_v.at[sl]], urows_v.at[sl, :], sem))
        copies.append(pltpu.async_copy(
            iemb_hbm.at[iidx_v.at[sl]], irows_v.at[sl, :], sem))
    copies.append(pltpu.async_copy(ibias_hbm.at[iidx_v], bias_v, sem))
    for cp in copies:
        cp.wait()

    iota16 = lax.iota(jnp.int32, 16)
    zeros16 = jnp.zeros((16,), jnp.int32)

    def rows16(g, carry):
        out16 = jnp.zeros((16,), jnp.float32)
        for k in range(16):
            r = g * 16 + k
            u0 = urows_v[r, pl.ds(0, 16)]
            u1 = urows_v[r, pl.ds(16, 16)]
            u2 = urows_v[r, pl.ds(32, 16)]
            u3 = urows_v[r, pl.ds(48, 16)]
            w0 = irows_v[r, pl.ds(0, 16)]
            w1 = irows_v[r, pl.ds(16, 16)]
            w2 = irows_v[r, pl.ds(32, 16)]
            w3 = irows_v[r, pl.ds(48, 16)]
            acc = (u0 * w0 + u1 * w1) + (u2 * w2 + u3 * w3)
            s = jnp.sum(acc)
            out16 = jnp.where(iota16 == k, s, out16)
        b16 = plsc.load_gather(bias_v, [g * 16 + iota16, zeros16])
        out_v[pl.ds(g * 16, 16)] = out16 + b16
        return carry

    lax.fori_loop(0, B_PER_W // 16, rows16, 0)

    pltpu.sync_copy(out_v, out_hbm.at[pl.ds(base, B_PER_W)])


@jax.jit
def kernel(user_ids, item_ids, user_embed, item_embed, item_bias):
    mesh = plsc.VectorSubcoreMesh(core_axis_name="c", subcore_axis_name="s")
    run = functools.partial(
        pl.kernel,
        mesh=mesh,
        compiler_params=pltpu.CompilerParams(
            needs_layout_passes=False, use_tc_tiling_on_sc=False),
        out_type=jax.ShapeDtypeStruct((BATCH,), jnp.float32),
        scratch_types=[
            pltpu.VMEM((B_PER_W,), jnp.int32),
            pltpu.VMEM((B_PER_W,), jnp.int32),
            pltpu.VMEM((B_PER_W, EMBED_DIM), jnp.float32),
            pltpu.VMEM((B_PER_W, EMBED_DIM), jnp.float32),
            pltpu.VMEM((B_PER_W, 1), jnp.float32),
            pltpu.VMEM((B_PER_W,), jnp.float32),
            pltpu.SemaphoreType.DMA,
        ],
    )(_dot_kernel)
    return run(user_ids, item_ids, user_embed, item_embed, item_bias)


# speedup vs baseline: 2.0135x; 1.7871x over previous
import functools

import jax
import jax.numpy as jnp
from jax import lax
from jax.experimental import pallas as pl
from jax.experimental.pallas import tpu as pltpu
from jax.experimental.pallas import tpu_sc as plsc

BATCH = 16384
EMBED_DIM = 64
NUM_WORKERS = 32
B_PER_W = BATCH // NUM_WORKERS
HALF = B_PER_W // 2


def _dot_kernel(uid_hbm, iid_hbm, uemb_hbm, iemb_hbm, ibias_hbm, out_hbm,
                uidx_v, iidx_v, urows_v, irows_v, bias_v, out_v, sem, bsem):
    wid = lax.axis_index("s") * 2 + lax.axis_index("c")
    base = wid * B_PER_W

    pltpu.sync_copy(uid_hbm.at[pl.ds(base, B_PER_W)], uidx_v)
    pltpu.sync_copy(iid_hbm.at[pl.ds(base, B_PER_W)], iidx_v)

    iota16 = lax.iota(jnp.int32, 16)
    zeros16 = jnp.zeros((16,), jnp.int32)


    for h in range(2):
        def issue(g, carry):
            uvec = uidx_v[pl.ds(h * HALF + g * 16, 16)]
            ivec = iidx_v[pl.ds(h * HALF + g * 16, 16)]
            for k in range(16):
                r = g * 16 + k
                iu = uvec[k]
                ii = ivec[k]
                pltpu.make_async_copy(
                    uemb_hbm.at[pl.ds(iu, 1), :],
                    urows_v.at[pl.ds(r, 1), :], sem,
                ).start()
                pltpu.make_async_copy(
                    iemb_hbm.at[pl.ds(ii, 1), :],
                    irows_v.at[pl.ds(r, 1), :], sem,
                ).start()
                pltpu.make_async_copy(
                    ibias_hbm.at[pl.ds(ii, 1)],
                    bias_v.at[pl.ds(r, 1), :], bsem,
                ).start()
            return carry

        lax.fori_loop(0, HALF // 16, issue, 0)

        def drain(r, carry):
            pltpu.make_async_copy(
                uemb_hbm.at[pl.ds(0, 1), :], urows_v.at[pl.ds(0, 1), :], sem,
            ).wait()
            pltpu.make_async_copy(
                iemb_hbm.at[pl.ds(0, 1), :], irows_v.at[pl.ds(0, 1), :], sem,
            ).wait()
            pltpu.make_async_copy(
                ibias_hbm.at[pl.ds(0, 1)], bias_v.at[pl.ds(0, 1), :], bsem,
            ).wait()
            return carry

        lax.fori_loop(0, HALF, drain, 0)

        def rows16(g, carry):
            out16 = jnp.zeros((16,), jnp.float32)
            for k in range(16):
                r = g * 16 + k
                u0 = urows_v[r, pl.ds(0, 16)]
                u1 = urows_v[r, pl.ds(16, 16)]
                u2 = urows_v[r, pl.ds(32, 16)]
                u3 = urows_v[r, pl.ds(48, 16)]
                w0 = irows_v[r, pl.ds(0, 16)]
                w1 = irows_v[r, pl.ds(16, 16)]
                w2 = irows_v[r, pl.ds(32, 16)]
                w3 = irows_v[r, pl.ds(48, 16)]
                acc = (u0 * w0 + u1 * w1) + (u2 * w2 + u3 * w3)
                s = jnp.sum(acc)
                out16 = jnp.where(iota16 == k, s, out16)
            b16 = plsc.load_gather(bias_v, [g * 16 + iota16, zeros16])
            out_v[pl.ds(h * HALF + g * 16, 16)] = out16 + b16
            return carry

        lax.fori_loop(0, HALF // 16, rows16, 0)

    pltpu.sync_copy(out_v, out_hbm.at[pl.ds(base, B_PER_W)])


@jax.jit
def kernel(user_ids, item_ids, user_embed, item_embed, item_bias):
    mesh = plsc.VectorSubcoreMesh(core_axis_name="c", subcore_axis_name="s")
    run = functools.partial(
        pl.kernel,
        mesh=mesh,
        compiler_params=pltpu.CompilerParams(
            needs_layout_passes=False, use_tc_tiling_on_sc=True),
        out_type=jax.ShapeDtypeStruct((BATCH,), jnp.float32),
        scratch_types=[
            pltpu.VMEM((B_PER_W,), jnp.int32),
            pltpu.VMEM((B_PER_W,), jnp.int32),
            pltpu.VMEM((HALF, EMBED_DIM), jnp.float32),
            pltpu.VMEM((HALF, EMBED_DIM), jnp.float32),
            pltpu.VMEM((HALF, 1), jnp.float32),
            pltpu.VMEM((B_PER_W,), jnp.float32),
            pltpu.SemaphoreType.DMA,
            pltpu.SemaphoreType.DMA,
        ],
    )(_dot_kernel)
    return run(user_ids, item_ids, user_embed, item_embed, item_bias)


# speedup vs baseline: 2.0212x; 1.0038x over previous
import functools

import jax
import jax.numpy as jnp
from jax import lax
from jax.experimental import pallas as pl
from jax.experimental.pallas import tpu as pltpu
from jax.experimental.pallas import tpu_sc as plsc

BATCH = 16384
EMBED_DIM = 64
NUM_WORKERS = 32
B_PER_W = BATCH // NUM_WORKERS
HALF = B_PER_W // 2


def _dot_kernel(uid_hbm, iid_hbm, uemb_hbm, iemb_hbm, ibias_hbm, out_hbm,
                uidx_v, iidx_v, urows_v, irows_v, bias_v, out_v, sem, bsem):
    wid = lax.axis_index("s") * 2 + lax.axis_index("c")
    base = wid * B_PER_W

    pltpu.sync_copy(uid_hbm.at[pl.ds(base, B_PER_W)], uidx_v)
    pltpu.sync_copy(iid_hbm.at[pl.ds(base, B_PER_W)], iidx_v)

    iota16 = lax.iota(jnp.int32, 16)
    zeros16 = jnp.zeros((16,), jnp.int32)


    for h in range(2):
        def issue(g, carry):
            uvec = uidx_v[pl.ds(h * HALF + g * 16, 16)]
            ivec = iidx_v[pl.ds(h * HALF + g * 16, 16)]
            for k in range(16):
                r = g * 16 + k
                iu = uvec[k]
                ii = ivec[k]
                pltpu.make_async_copy(
                    uemb_hbm.at[pl.ds(iu, 1), :],
                    urows_v.at[pl.ds(r, 1), :], sem,
                ).start()
                pltpu.make_async_copy(
                    iemb_hbm.at[pl.ds(ii, 1), :],
                    irows_v.at[pl.ds(r, 1), :], sem,
                ).start()
                pltpu.make_async_copy(
                    ibias_hbm.at[pl.ds(ii, 1)],
                    bias_v.at[pl.ds(r, 1), :], bsem,
                ).start()
            return carry

        lax.fori_loop(0, HALF // 16, issue, 0)

        pltpu.make_async_copy(
            uemb_hbm.at[pl.ds(0, HALF), :], urows_v, sem).wait()
        pltpu.make_async_copy(
            iemb_hbm.at[pl.ds(0, HALF), :], irows_v, sem).wait()
        pltpu.make_async_copy(
            ibias_hbm.at[pl.ds(0, HALF)], bias_v, bsem).wait()

        def rows16(g, carry):
            out16 = jnp.zeros((16,), jnp.float32)
            for k in range(16):
                r = g * 16 + k
                u0 = urows_v[r, pl.ds(0, 16)]
                u1 = urows_v[r, pl.ds(16, 16)]
                u2 = urows_v[r, pl.ds(32, 16)]
                u3 = urows_v[r, pl.ds(48, 16)]
                w0 = irows_v[r, pl.ds(0, 16)]
                w1 = irows_v[r, pl.ds(16, 16)]
                w2 = irows_v[r, pl.ds(32, 16)]
                w3 = irows_v[r, pl.ds(48, 16)]
                acc = (u0 * w0 + u1 * w1) + (u2 * w2 + u3 * w3)
                s = jnp.sum(acc)
                out16 = jnp.where(iota16 == k, s, out16)
            b16 = plsc.load_gather(bias_v, [g * 16 + iota16, zeros16])
            out_v[pl.ds(h * HALF + g * 16, 16)] = out16 + b16
            return carry

        lax.fori_loop(0, HALF // 16, rows16, 0)

    pltpu.sync_copy(out_v, out_hbm.at[pl.ds(base, B_PER_W)])


@jax.jit
def kernel(user_ids, item_ids, user_embed, item_embed, item_bias):
    mesh = plsc.VectorSubcoreMesh(core_axis_name="c", subcore_axis_name="s")
    run = functools.partial(
        pl.kernel,
        mesh=mesh,
        compiler_params=pltpu.CompilerParams(
            needs_layout_passes=False, use_tc_tiling_on_sc=True),
        out_type=jax.ShapeDtypeStruct((BATCH,), jnp.float32),
        scratch_types=[
            pltpu.VMEM((B_PER_W,), jnp.int32),
            pltpu.VMEM((B_PER_W,), jnp.int32),
            pltpu.VMEM((HALF, EMBED_DIM), jnp.float32),
            pltpu.VMEM((HALF, EMBED_DIM), jnp.float32),
            pltpu.VMEM((HALF, 1), jnp.float32),
            pltpu.VMEM((B_PER_W,), jnp.float32),
            pltpu.SemaphoreType.DMA,
            pltpu.SemaphoreType.DMA,
        ],
    )(_dot_kernel)
    return run(user_ids, item_ids, user_embed, item_embed, item_bias)
